# Initial kernel scaffold; baseline (speedup 1.0000x reference)
import jax
import jax.numpy as jnp
from jax.experimental import pallas as pl


def kernel(x, custom_indices, custom_table, regular_table, W, b):
    raise NotImplementedError("write your pallas kernel here")


# speedup vs baseline: 8.7538x; 8.7538x over previous
import functools

import jax
import jax.numpy as jnp
from jax import lax
from jax.experimental import pallas as pl
from jax.experimental.pallas import tpu as pltpu
from jax.experimental.pallas import tpu_sc as plsc

HEAD = 4096
D = 64
_HEAD_BLK = 512


def _head_body(ci_ref, ct_ref, rt_ref, w_ref, b_ref, out_ref):
    rows = ct_ref.shape[0]
    base = pl.program_id(0) * rows
    row_ids = base + lax.broadcasted_iota(jnp.int32, (rows, 1), 0)
    member = jnp.any(row_ids == ci_ref[...], axis=1, keepdims=True)
    proj = lax.dot_general(ct_ref[...], w_ref[...], (((1,), (1,)), ((), ())),
                           preferred_element_type=jnp.float32)
    out_ref[...] = jnp.where(member, proj, rt_ref[...]) + b_ref[...]


def _build_head(custom_indices, custom_table, regular_head, W, b):
    n_ci = custom_indices.shape[0]
    ci2 = custom_indices.reshape(1, n_ci).astype(jnp.int32)
    b2 = b.reshape(1, D)
    return pl.pallas_call(
        _head_body,
        grid=(HEAD // _HEAD_BLK,),
        in_specs=[
            pl.BlockSpec((1, n_ci), lambda i: (0, 0)),
            pl.BlockSpec((_HEAD_BLK, D), lambda i: (i, 0)),
            pl.BlockSpec((_HEAD_BLK, D), lambda i: (i, 0)),
            pl.BlockSpec((D, D), lambda i: (0, 0)),
            pl.BlockSpec((1, D), lambda i: (0, 0)),
        ],
        out_specs=pl.BlockSpec((_HEAD_BLK, D), lambda i: (i, 0)),
        out_shape=jax.ShapeDtypeStruct((HEAD, D), jnp.float32),
    )(ci2, custom_table, regular_head, W, b2)


def _sc_gather(table, idx_flat):
    info = plsc.get_sparse_core_info()
    nc, ns = info.num_cores, info.num_subcores
    nw = nc * ns
    n = idx_flat.shape[0]
    tpw = n // nw
    chunk = 512
    nch = tpw // chunk

    mesh = plsc.VectorSubcoreMesh(core_axis_name="c", subcore_axis_name="s")

    @functools.partial(
        pl.kernel, mesh=mesh,
        out_type=jax.ShapeDtypeStruct((n, D), jnp.float32),
        compiler_params=pltpu.CompilerParams(use_tc_tiling_on_sc=False),
        scratch_types=[
            pltpu.VMEM((chunk,), jnp.int32),
            pltpu.VMEM((chunk, D), jnp.float32),
            pltpu.SemaphoreType.DMA,
        ],
    )
    def k(table_hbm, idx_hbm, out_hbm, idx_v, rows_v, sem):
        wid = lax.axis_index("s") * nc + lax.axis_index("c")
        base = wid * tpw

        def body(g, carry):
            off = base + g * chunk
            pltpu.sync_copy(idx_hbm.at[pl.ds(off, chunk)], idx_v)
            pltpu.async_copy(table_hbm.at[idx_v], rows_v, sem).wait()
            pltpu.sync_copy(rows_v, out_hbm.at[pl.ds(off, chunk)])
            return carry

        lax.fori_loop(0, nch, body, 0)

    return k(table, idx_flat)


def kernel(x, custom_indices, custom_table, regular_table, W, b):
    head = _build_head(custom_indices, custom_table, regular_table[:HEAD], W, b)
    merged = jnp.concatenate([head, regular_table[HEAD:] + b], axis=0)
    idx_flat = x.reshape(-1).astype(jnp.int32)
    out = _sc_gather(merged, idx_flat)
    return out.reshape(x.shape + (D,))


# speedup vs baseline: 9.1262x; 1.0425x over previous
import functools

import jax
import jax.numpy as jnp
from jax import lax
from jax.experimental import pallas as pl
from jax.experimental.pallas import tpu as pltpu
from jax.experimental.pallas import tpu_sc as plsc

HEAD = 4096
D = 64
_HEAD_BLK = 512


def _head_body(ci_ref, ct_ref, rt_ref, w_ref, b_ref, out_ref):
    rows = ct_ref.shape[0]
    base = pl.program_id(0) * rows
    row_ids = base + lax.broadcasted_iota(jnp.int32, (rows, 1), 0)
    member = jnp.any(row_ids == ci_ref[...], axis=1, keepdims=True)
    proj = lax.dot_general(ct_ref[...], w_ref[...], (((1,), (1,)), ((), ())),
                           preferred_element_type=jnp.float32)
    out_ref[...] = jnp.where(member, proj, rt_ref[...]) + b_ref[...]


def _build_head(custom_indices, custom_table, regular_head, W, b):
    n_ci = custom_indices.shape[0]
    ci2 = custom_indices.reshape(1, n_ci).astype(jnp.int32)
    b2 = b.reshape(1, D)
    return pl.pallas_call(
        _head_body,
        grid=(HEAD // _HEAD_BLK,),
        in_specs=[
            pl.BlockSpec((1, n_ci), lambda i: (0, 0)),
            pl.BlockSpec((_HEAD_BLK, D), lambda i: (i, 0)),
            pl.BlockSpec((_HEAD_BLK, D), lambda i: (i, 0)),
            pl.BlockSpec((D, D), lambda i: (0, 0)),
            pl.BlockSpec((1, D), lambda i: (0, 0)),
        ],
        out_specs=pl.BlockSpec((_HEAD_BLK, D), lambda i: (i, 0)),
        out_shape=jax.ShapeDtypeStruct((HEAD, D), jnp.float32),
    )(ci2, custom_table, regular_head, W, b2)


def _sc_gather(table, idx_flat):
    info = plsc.get_sparse_core_info()
    nc, ns = info.num_cores, info.num_subcores
    nw = nc * ns
    n = idx_flat.shape[0]
    tpw = n // nw
    chunk = 512
    nch = tpw // chunk

    mesh = plsc.VectorSubcoreMesh(core_axis_name="c", subcore_axis_name="s")

    @functools.partial(
        pl.kernel, mesh=mesh,
        out_type=jax.ShapeDtypeStruct((n, D), jnp.float32),
        compiler_params=pltpu.CompilerParams(use_tc_tiling_on_sc=False),
        scratch_types=[
            pltpu.VMEM((chunk,), jnp.int32),
            pltpu.VMEM((chunk,), jnp.int32),
            pltpu.VMEM((chunk, D), jnp.float32),
            pltpu.VMEM((chunk, D), jnp.float32),
            pltpu.SemaphoreType.DMA,
            pltpu.SemaphoreType.DMA,
            pltpu.SemaphoreType.DMA,
            pltpu.SemaphoreType.DMA,
        ],
    )
    def k(table_hbm, idx_hbm, out_hbm, idx0, idx1, rows0, rows1,
          g0, g1, w0, w1):
        wid = lax.axis_index("s") * nc + lax.axis_index("c")
        base = wid * tpw
        idx_v = (idx0, idx1)
        rows_v = (rows0, rows1)
        gsem = (g0, g1)
        wsem = (w0, w1)

        def gather_start(g, p):
            pltpu.sync_copy(idx_hbm.at[pl.ds(base + g * chunk, chunk)],
                            idx_v[p])
            pltpu.async_copy(table_hbm.at[idx_v[p]], rows_v[p], gsem[p])

        def gather_wait(p):
            pltpu.make_async_copy(table_hbm.at[idx_v[p]], rows_v[p],
                                  gsem[p]).wait()

        def write_start(g, p):
            pltpu.async_copy(rows_v[p], out_hbm.at[pl.ds(base + g * chunk,
                                                         chunk)], wsem[p])

        def write_wait(p):
            pltpu.make_async_copy(rows_v[p], out_hbm.at[pl.ds(base, chunk)],
                                  wsem[p]).wait()

        gather_start(0, 0)
        gather_start(1, 1)
        gather_wait(0)
        write_start(0, 0)

        def body(i2, carry):
            i = 2 * i2 + 1
            write_wait(0)
            gather_wait(1)
            write_start(i, 1)
            gather_start(i + 1, 0)
            write_wait(1)
            gather_wait(0)
            write_start(i + 1, 0)
            gather_start(i + 2, 1)
            return carry

        lax.fori_loop(0, (nch - 1) // 2, body, 0)
        gather_wait(1)
        write_start(nch - 1, 1)
        write_wait(0)
        write_wait(1)

    return k(table, idx_flat)


def kernel(x, custom_indices, custom_table, regular_table, W, b):
    head = _build_head(custom_indices, custom_table, regular_table[:HEAD], W, b)
    merged = jnp.concatenate([head, regular_table[HEAD:] + b], axis=0)
    idx_flat = x.reshape(-1).astype(jnp.int32)
    out = _sc_gather(merged, idx_flat)
    return out.reshape(x.shape + (D,))


# speedup vs baseline: 9.3134x; 1.0205x over previous
import functools

import jax
import jax.numpy as jnp
from jax import lax
from jax.experimental import pallas as pl
from jax.experimental.pallas import tpu as pltpu
from jax.experimental.pallas import tpu_sc as plsc

HEAD = 4096
D = 64
_HEAD_BLK = 512


def _head_body(ci_ref, ct_ref, rt_ref, w_ref, b_ref, out_ref):
    rows = ct_ref.shape[0]
    base = pl.program_id(0) * rows
    row_ids = base + lax.broadcasted_iota(jnp.int32, (rows, 1), 0)
    member = jnp.any(row_ids == ci_ref[...], axis=1, keepdims=True)
    proj = lax.dot_general(ct_ref[...], w_ref[...], (((1,), (1,)), ((), ())),
                           preferred_element_type=jnp.float32)
    out_ref[...] = jnp.where(member, proj, rt_ref[...]) + b_ref[...]


def _build_head(custom_indices, custom_table, regular_head, W, b):
    n_ci = custom_indices.shape[0]
    ci2 = custom_indices.reshape(1, n_ci).astype(jnp.int32)
    b2 = b.reshape(1, D)
    return pl.pallas_call(
        _head_body,
        grid=(HEAD // _HEAD_BLK,),
        in_specs=[
            pl.BlockSpec((1, n_ci), lambda i: (0, 0)),
            pl.BlockSpec((_HEAD_BLK, D), lambda i: (i, 0)),
            pl.BlockSpec((_HEAD_BLK, D), lambda i: (i, 0)),
            pl.BlockSpec((D, D), lambda i: (0, 0)),
            pl.BlockSpec((1, D), lambda i: (0, 0)),
        ],
        out_specs=pl.BlockSpec((_HEAD_BLK, D), lambda i: (i, 0)),
        out_shape=jax.ShapeDtypeStruct((HEAD, D), jnp.float32),
    )(ci2, custom_table, regular_head, W, b2)


def _sc_gather(table, idx_flat):
    info = plsc.get_sparse_core_info()
    nc, ns = info.num_cores, info.num_subcores
    nw = nc * ns
    n = idx_flat.shape[0]
    tpw = n // nw
    chunk = 400
    nch = tpw // chunk
    nbuf = 4

    mesh = plsc.VectorSubcoreMesh(core_axis_name="c", subcore_axis_name="s")

    @functools.partial(
        pl.kernel, mesh=mesh,
        out_type=jax.ShapeDtypeStruct((n, D), jnp.float32),
        compiler_params=pltpu.CompilerParams(use_tc_tiling_on_sc=False),
        scratch_types=[pltpu.VMEM((tpw,), jnp.int32)]
                      + [pltpu.VMEM((chunk, D), jnp.float32)] * nbuf
                      + [pltpu.SemaphoreType.DMA] * nbuf,
    )
    def k(table_hbm, idx_hbm, out_hbm, idx_v, r0, r1, r2, r3,
          g0, g1, g2, g3):
        rows = (r0, r1, r2, r3)
        gsem = (g0, g1, g2, g3)
        wid = lax.axis_index("s") * nc + lax.axis_index("c")
        base = wid * tpw

        pltpu.sync_copy(idx_hbm.at[pl.ds(base, tpw)], idx_v)

        def gstart(g, p):
            pltpu.async_copy(
                table_hbm.at[idx_v.at[pl.ds(g * chunk, chunk)]],
                rows[p], gsem[p])

        def gwait(p):
            pltpu.make_async_copy(
                table_hbm.at[idx_v.at[pl.ds(0, chunk)]], rows[p],
                gsem[p]).wait()

        for p in range(nbuf):
            gstart(p, p)

        def body(i, carry):
            for p in range(nbuf):
                g = i * nbuf + p
                gwait(p)
                pltpu.sync_copy(rows[p],
                                out_hbm.at[pl.ds(base + g * chunk, chunk)])
                gstart(g + nbuf, p)
            return carry

        lax.fori_loop(0, nch // nbuf - 1, body, 0)

        for p in range(nbuf):
            g = nch - nbuf + p
            gwait(p)
            pltpu.sync_copy(rows[p],
                            out_hbm.at[pl.ds(base + g * chunk, chunk)])

    return k(table, idx_flat)


def kernel(x, custom_indices, custom_table, regular_table, W, b):
    head = _build_head(custom_indices, custom_table, regular_table[:HEAD], W, b)
    merged = jnp.concatenate([head, regular_table[HEAD:] + b], axis=0)
    idx_flat = x.reshape(-1).astype(jnp.int32)
    out = _sc_gather(merged, idx_flat)
    return out.reshape(x.shape + (D,))
